# Initial kernel scaffold; baseline (speedup 1.0000x reference)
import jax
import jax.numpy as jnp
from jax.experimental import pallas as pl


def kernel(inputs, edge_index, edge_attr, W_gcn, b_gcn, W_fc, b_fc):
    raise NotImplementedError("write your pallas kernel here")


# speedup vs baseline: 4.4779x; 4.4779x over previous
import functools

import jax
import jax.numpy as jnp
from jax import lax
from jax.experimental import pallas as pl
from jax.experimental.pallas import tpu as pltpu
from jax.experimental.pallas import tpu_sc as plsc

N_NODES = 10000
N_EDGES = 320000
FEATS = 128

NC = 2
NS = 16
NW = NC * NS
EPW = N_EDGES // NW
CH = 80
NCH = EPW // CH
RPT = 624
CHR = 104
TAIL = N_NODES - NS * RPT
LG = FEATS // 16


def _mm_xw_kernel(x_ref, w_ref, o_ref):
    o_ref[...] = jnp.dot(x_ref[...], w_ref[...],
                         preferred_element_type=jnp.float32)


def _mm_fc_kernel(p_ref, w_ref, bg_ref, bf_ref, o_ref):
    h = p_ref[0] + p_ref[1] + bg_ref[...]
    o_ref[...] = lax.dot_general(
        h, w_ref[...], (((1,), (1,)), ((), ())),
        preferred_element_type=jnp.float32) + bf_ref[...]


def _sc_scatter_body(xw_hbm, src_hbm, dst_hbm, attr_hbm, out_hbm,
                     src_v, dst_v, attr_v, rows_v, zbuf, acc, sem):
    c = lax.axis_index("c")
    s = lax.axis_index("s")
    wid = s * NC + c

    zero16 = jnp.zeros((16,), jnp.float32)

    def zrow(i, carry):
        for j in range(LG):
            zbuf[i, pl.ds(16 * j, 16)] = zero16
        return carry

    lax.fori_loop(0, CHR, zrow, 0)
    for k in range(RPT // CHR):
        pltpu.sync_copy(zbuf, acc.at[pl.ds(s * RPT + k * CHR, CHR)])

    @pl.when(s == NS - 1)
    def _zero_tail():
        pltpu.sync_copy(zbuf.at[pl.ds(0, TAIL)],
                        acc.at[pl.ds(NS * RPT, TAIL)])

    plsc.subcore_barrier()

    base = wid * EPW

    def chunk(k, carry):
        off = base + k * CH
        pltpu.sync_copy(src_hbm.at[pl.ds(off, CH)], src_v)
        pltpu.sync_copy(dst_hbm.at[pl.ds(off, CH)], dst_v)
        pltpu.sync_copy(attr_hbm.at[pl.ds(off, CH)], attr_v)
        pltpu.async_copy(xw_hbm.at[src_v], rows_v, sem).wait()

        def scale16(g, inner):
            a_vec = attr_v[pl.ds(g * 16, 16)]
            for i in range(16):
                a = a_vec[i]
                e = g * 16 + i
                for j in range(LG):
                    rows_v[e, pl.ds(16 * j, 16)] = (
                        rows_v[e, pl.ds(16 * j, 16)] * a)
            return inner

        lax.fori_loop(0, CH // 16, scale16, 0)
        pltpu.sync_copy(rows_v, acc.at[dst_v], add=True)
        return carry

    lax.fori_loop(0, NCH, chunk, 0)
    plsc.subcore_barrier()

    for k in range(RPT // CHR):
        start = s * RPT + k * CHR
        pltpu.sync_copy(acc.at[pl.ds(start, CHR)],
                        out_hbm.at[c, pl.ds(start, CHR)])

    @pl.when(s == NS - 1)
    def _copy_tail():
        pltpu.sync_copy(acc.at[pl.ds(NS * RPT, TAIL)],
                        out_hbm.at[c, pl.ds(NS * RPT, TAIL)])


_sc_scatter = functools.partial(
    pl.kernel,
    mesh=plsc.VectorSubcoreMesh(core_axis_name="c", subcore_axis_name="s"),
    out_type=jax.ShapeDtypeStruct((NC, N_NODES, FEATS), jnp.float32),
    scratch_types=[
        pltpu.VMEM((CH,), jnp.int32),
        pltpu.VMEM((CH,), jnp.int32),
        pltpu.VMEM((CH,), jnp.float32),
        pltpu.VMEM((CH, FEATS), jnp.float32),
        pltpu.VMEM((CHR, FEATS), jnp.float32),
        pltpu.VMEM_SHARED((N_NODES, FEATS), jnp.float32),
        pltpu.SemaphoreType.DMA,
    ],
)(_sc_scatter_body)


def kernel(inputs, edge_index, edge_attr, W_gcn, b_gcn, W_fc, b_fc):
    src = edge_index[0].astype(jnp.int32)
    dst = edge_index[1].astype(jnp.int32)
    attr = edge_attr.astype(jnp.float32)

    blk = 1000
    grid = N_NODES // blk
    xw = pl.pallas_call(
        _mm_xw_kernel,
        grid=(grid,),
        in_specs=[
            pl.BlockSpec((blk, FEATS), lambda i: (i, 0)),
            pl.BlockSpec((FEATS, FEATS), lambda i: (0, 0)),
        ],
        out_specs=pl.BlockSpec((blk, FEATS), lambda i: (i, 0)),
        out_shape=jax.ShapeDtypeStruct((N_NODES, FEATS), jnp.float32),
    )(inputs, W_gcn)

    partials = _sc_scatter(xw, src, dst, attr)

    preds = pl.pallas_call(
        _mm_fc_kernel,
        grid=(grid,),
        in_specs=[
            pl.BlockSpec((NC, blk, FEATS), lambda i: (0, i, 0)),
            pl.BlockSpec((FEATS, FEATS), lambda i: (0, 0)),
            pl.BlockSpec((1, FEATS), lambda i: (0, 0)),
            pl.BlockSpec((1, FEATS), lambda i: (0, 0)),
        ],
        out_specs=pl.BlockSpec((blk, FEATS), lambda i: (i, 0)),
        out_shape=jax.ShapeDtypeStruct((N_NODES, FEATS), jnp.float32),
    )(partials, W_fc, b_gcn.reshape(1, FEATS), b_fc.reshape(1, FEATS))

    return preds


# speedup vs baseline: 6.7268x; 1.5022x over previous
import functools

import jax
import jax.numpy as jnp
from jax import lax
from jax.experimental import pallas as pl
from jax.experimental.pallas import tpu as pltpu
from jax.experimental.pallas import tpu_sc as plsc

N_NODES = 10000
N_EDGES = 320000
FEATS = 128

NC = 2
NS = 16
NW = NC * NS
EPW = N_EDGES // NW
CH = 80
NCH = EPW // CH
RPT = 624
CHR = 104
TAIL = N_NODES - NS * RPT
LG = FEATS // 16


def _mm_xw_kernel(x_ref, w_ref, o_ref):
    o_ref[...] = jnp.dot(x_ref[...], w_ref[...],
                         preferred_element_type=jnp.float32)


def _mm_fc_kernel(p_ref, w_ref, bg_ref, bf_ref, o_ref):
    h = p_ref[0] + p_ref[1] + bg_ref[...]
    o_ref[...] = lax.dot_general(
        h, w_ref[...], (((1,), (1,)), ((), ())),
        preferred_element_type=jnp.float32) + bf_ref[...]


def _sc_scatter_body(xw_hbm, src_hbm, dst_hbm, attr_hbm, out_hbm,
                     src_v, dst_v, attr_v, rows_v, acc, sem):
    c = lax.axis_index("c")
    s = lax.axis_index("s")
    wid = s * NC + c

    zero16 = jnp.zeros((16,), jnp.float32)

    def zrow(i, carry):
        for j in range(LG):
            rows_v[i, pl.ds(16 * j, 16)] = zero16
        return carry

    lax.fori_loop(0, CH, zrow, 0)
    for k in range(7):
        pltpu.sync_copy(rows_v, acc.at[pl.ds(s * RPT + k * CH, CH)])
    pltpu.sync_copy(rows_v.at[pl.ds(0, 64)],
                    acc.at[pl.ds(s * RPT + 7 * CH, 64)])

    @pl.when(s == NS - 1)
    def _zero_tail():
        pltpu.sync_copy(rows_v.at[pl.ds(0, TAIL)],
                        acc.at[pl.ds(NS * RPT, TAIL)])

    plsc.subcore_barrier()

    base = wid * EPW
    pltpu.sync_copy(src_hbm.at[pl.ds(base, EPW)], src_v)
    pltpu.sync_copy(dst_hbm.at[pl.ds(base, EPW)], dst_v)
    pltpu.sync_copy(attr_hbm.at[pl.ds(base, EPW)], attr_v)

    def chunk(k, carry):
        off = k * CH
        pltpu.async_copy(xw_hbm.at[src_v.at[pl.ds(off, CH)]],
                         rows_v, sem).wait()

        def scale16(g, inner):
            a_vec = attr_v[pl.ds(off + g * 16, 16)]
            for i in range(16):
                a = a_vec[i]
                for j in range(LG):
                    rows_v[g * 16 + i, pl.ds(16 * j, 16)] = (
                        rows_v[g * 16 + i, pl.ds(16 * j, 16)] * a)
            return inner

        lax.fori_loop(0, CH // 16, scale16, 0)
        pltpu.sync_copy(rows_v, acc.at[dst_v.at[pl.ds(off, CH)]], add=True)
        return carry

    lax.fori_loop(0, NCH, chunk, 0)
    plsc.subcore_barrier()

    for k in range(RPT // CHR):
        start = s * RPT + k * CHR
        pltpu.sync_copy(acc.at[pl.ds(start, CHR)],
                        out_hbm.at[c, pl.ds(start, CHR)])

    @pl.when(s == NS - 1)
    def _copy_tail():
        pltpu.sync_copy(acc.at[pl.ds(NS * RPT, TAIL)],
                        out_hbm.at[c, pl.ds(NS * RPT, TAIL)])


_sc_scatter = functools.partial(
    pl.kernel,
    mesh=plsc.VectorSubcoreMesh(core_axis_name="c", subcore_axis_name="s"),
    out_type=jax.ShapeDtypeStruct((NC, N_NODES, FEATS), jnp.float32),
    scratch_types=[
        pltpu.VMEM((EPW,), jnp.int32),
        pltpu.VMEM((EPW,), jnp.int32),
        pltpu.VMEM((EPW,), jnp.float32),
        pltpu.VMEM((CH, FEATS), jnp.float32),
        pltpu.VMEM_SHARED((N_NODES, FEATS), jnp.float32),
        pltpu.SemaphoreType.DMA,
    ],
)(_sc_scatter_body)


def kernel(inputs, edge_index, edge_attr, W_gcn, b_gcn, W_fc, b_fc):
    src = edge_index[0].astype(jnp.int32)
    dst = edge_index[1].astype(jnp.int32)
    attr = edge_attr.astype(jnp.float32)

    blk = 1000
    grid = N_NODES // blk
    xw = pl.pallas_call(
        _mm_xw_kernel,
        grid=(grid,),
        in_specs=[
            pl.BlockSpec((blk, FEATS), lambda i: (i, 0)),
            pl.BlockSpec((FEATS, FEATS), lambda i: (0, 0)),
        ],
        out_specs=pl.BlockSpec((blk, FEATS), lambda i: (i, 0)),
        out_shape=jax.ShapeDtypeStruct((N_NODES, FEATS), jnp.float32),
    )(inputs, W_gcn)

    partials = _sc_scatter(xw, src, dst, attr)

    preds = pl.pallas_call(
        _mm_fc_kernel,
        grid=(grid,),
        in_specs=[
            pl.BlockSpec((NC, blk, FEATS), lambda i: (0, i, 0)),
            pl.BlockSpec((FEATS, FEATS), lambda i: (0, 0)),
            pl.BlockSpec((1, FEATS), lambda i: (0, 0)),
            pl.BlockSpec((1, FEATS), lambda i: (0, 0)),
        ],
        out_specs=pl.BlockSpec((blk, FEATS), lambda i: (i, 0)),
        out_shape=jax.ShapeDtypeStruct((N_NODES, FEATS), jnp.float32),
    )(partials, W_fc, b_gcn.reshape(1, FEATS), b_fc.reshape(1, FEATS))

    return preds


# speedup vs baseline: 9.9527x; 1.4796x over previous
import functools

import jax
import jax.numpy as jnp
from jax import lax
from jax.experimental import pallas as pl
from jax.experimental.pallas import tpu as pltpu
from jax.experimental.pallas import tpu_sc as plsc

N_NODES = 10000
N_EDGES = 320000
FEATS = 128

NC = 2
NS = 16
NW = NC * NS
EPW = N_EDGES // NW
CH = 80
NCH = EPW // CH
RPT = 624
CHR = 104
TAIL = N_NODES - NS * RPT
LG = FEATS // 16


def _mm_xw_kernel(x_ref, w_ref, o_ref):
    o_ref[...] = jnp.dot(x_ref[...], w_ref[...],
                         preferred_element_type=jnp.float32)


def _mm_fc_kernel(p_ref, w_ref, bg_ref, bf_ref, o_ref):
    h = p_ref[0] + p_ref[1] + bg_ref[...]
    o_ref[...] = lax.dot_general(
        h, w_ref[...], (((1,), (1,)), ((), ())),
        preferred_element_type=jnp.float32) + bf_ref[...]


def _sc_scatter_body(xw_hbm, src_hbm, dst_hbm, attr_hbm, out_hbm,
                     src_v, dst_v, attr_v, rows0, rows1, acc, sem0, sem1):
    rows_v = rows0
    c = lax.axis_index("c")
    s = lax.axis_index("s")
    wid = s * NC + c

    zero16 = jnp.zeros((16,), jnp.float32)

    def zrow(i, carry):
        for j in range(LG):
            rows_v[i, pl.ds(16 * j, 16)] = zero16
        return carry

    lax.fori_loop(0, CH, zrow, 0)
    for k in range(7):
        pltpu.sync_copy(rows_v, acc.at[pl.ds(s * RPT + k * CH, CH)])
    pltpu.sync_copy(rows_v.at[pl.ds(0, 64)],
                    acc.at[pl.ds(s * RPT + 7 * CH, 64)])

    @pl.when(s == NS - 1)
    def _zero_tail():
        pltpu.sync_copy(rows_v.at[pl.ds(0, TAIL)],
                        acc.at[pl.ds(NS * RPT, TAIL)])

    plsc.subcore_barrier()

    base = wid * EPW
    pltpu.sync_copy(src_hbm.at[pl.ds(base, EPW)], src_v)
    pltpu.sync_copy(dst_hbm.at[pl.ds(base, EPW)], dst_v)
    pltpu.sync_copy(attr_hbm.at[pl.ds(base, EPW)], attr_v)

    bufs = (rows0, rows1)
    sems = (sem0, sem1)

    def _gather(k, buf, sem):
        return pltpu.async_copy(xw_hbm.at[src_v.at[pl.ds(k * CH, CH)]],
                                buf, sem)

    def _step(k, b, issue_next):
        buf, sem = bufs[b], sems[b]
        pltpu.make_async_copy(xw_hbm.at[src_v.at[pl.ds(k * CH, CH)]],
                              buf, sem).wait()
        if issue_next:
            _gather(k + 1, bufs[b ^ 1], sems[b ^ 1])
        off = k * CH

        def scale16(g, inner):
            a_vec = attr_v[pl.ds(off + g * 16, 16)]
            for i in range(16):
                a = a_vec[i]
                for j in range(LG):
                    buf[g * 16 + i, pl.ds(16 * j, 16)] = (
                        buf[g * 16 + i, pl.ds(16 * j, 16)] * a)
            return inner

        lax.fori_loop(0, CH // 16, scale16, 0)
        pltpu.sync_copy(buf, acc.at[dst_v.at[pl.ds(off, CH)]], add=True)

    _gather(0, rows0, sem0)

    def pair(k2, carry):
        _step(k2 * 2, 0, True)
        _step(k2 * 2 + 1, 1, True)
        return carry

    lax.fori_loop(0, (NCH - 1) // 2, pair, 0)
    _step(NCH - 1, 0, False)
    plsc.subcore_barrier()

    for k in range(RPT // CHR):
        start = s * RPT + k * CHR
        pltpu.sync_copy(acc.at[pl.ds(start, CHR)],
                        out_hbm.at[c, pl.ds(start, CHR)])

    @pl.when(s == NS - 1)
    def _copy_tail():
        pltpu.sync_copy(acc.at[pl.ds(NS * RPT, TAIL)],
                        out_hbm.at[c, pl.ds(NS * RPT, TAIL)])


_sc_scatter = functools.partial(
    pl.kernel,
    mesh=plsc.VectorSubcoreMesh(core_axis_name="c", subcore_axis_name="s"),
    out_type=jax.ShapeDtypeStruct((NC, N_NODES, FEATS), jnp.float32),
    scratch_types=[
        pltpu.VMEM((EPW,), jnp.int32),
        pltpu.VMEM((EPW,), jnp.int32),
        pltpu.VMEM((EPW,), jnp.float32),
        pltpu.VMEM((CH, FEATS), jnp.float32),
        pltpu.VMEM((CH, FEATS), jnp.float32),
        pltpu.VMEM_SHARED((N_NODES, FEATS), jnp.float32),
        pltpu.SemaphoreType.DMA,
        pltpu.SemaphoreType.DMA,
    ],
)(_sc_scatter_body)


def kernel(inputs, edge_index, edge_attr, W_gcn, b_gcn, W_fc, b_fc):
    src = edge_index[0].astype(jnp.int32)
    dst = edge_index[1].astype(jnp.int32)
    attr = edge_attr.astype(jnp.float32)

    blk = 1000
    grid = N_NODES // blk
    xw = pl.pallas_call(
        _mm_xw_kernel,
        grid=(grid,),
        in_specs=[
            pl.BlockSpec((blk, FEATS), lambda i: (i, 0)),
            pl.BlockSpec((FEATS, FEATS), lambda i: (0, 0)),
        ],
        out_specs=pl.BlockSpec((blk, FEATS), lambda i: (i, 0)),
        out_shape=jax.ShapeDtypeStruct((N_NODES, FEATS), jnp.float32),
    )(inputs, W_gcn)

    partials = _sc_scatter(xw, src, dst, attr)

    preds = pl.pallas_call(
        _mm_fc_kernel,
        grid=(grid,),
        in_specs=[
            pl.BlockSpec((NC, blk, FEATS), lambda i: (0, i, 0)),
            pl.BlockSpec((FEATS, FEATS), lambda i: (0, 0)),
            pl.BlockSpec((1, FEATS), lambda i: (0, 0)),
            pl.BlockSpec((1, FEATS), lambda i: (0, 0)),
        ],
        out_specs=pl.BlockSpec((blk, FEATS), lambda i: (i, 0)),
        out_shape=jax.ShapeDtypeStruct((N_NODES, FEATS), jnp.float32),
    )(partials, W_fc, b_gcn.reshape(1, FEATS), b_fc.reshape(1, FEATS))

    return preds
